# speedup vs baseline: 4.0943x; 4.0943x over previous
import jax
import jax.numpy as jnp
from jax.experimental import pallas as pl


def _split_body(x_ref, lo_ref, mid_ref, hi_ref):
    lo_ref[...] = x_ref[:, :, 0:48, :]
    mid_ref[...] = x_ref[:, 0, 48:96, :]
    hi_ref[...] = x_ref[:, :, 96:128, :]


def kernel(x, idx_low, idx_mid, idx_high):
    B, _, R, C = x.shape
    out_shape = (
        jax.ShapeDtypeStruct((B, 1, 48, C), x.dtype),
        jax.ShapeDtypeStruct((B, 48, C), x.dtype),
        jax.ShapeDtypeStruct((B, 1, 32, C), x.dtype),
    )
    return pl.pallas_call(
        _split_body,
        grid=(B,),
        in_specs=[pl.BlockSpec((1, 1, R, C), lambda b: (b, 0, 0, 0))],
        out_specs=(
            pl.BlockSpec((1, 1, 48, C), lambda b: (b, 0, 0, 0)),
            pl.BlockSpec((1, 48, C), lambda b: (b, 0, 0)),
            pl.BlockSpec((1, 1, 32, C), lambda b: (b, 0, 0, 0)),
        ),
        out_shape=out_shape,
    )(x)
